# Initial kernel scaffold; baseline (speedup 1.0000x reference)
import jax
import jax.numpy as jnp
from jax.experimental import pallas as pl


def kernel(x, y, k, n_splits):
    raise NotImplementedError("write your pallas kernel here")


# speedup vs baseline: 1.4900x; 1.4900x over previous
import jax
import jax.numpy as jnp
from jax.experimental import pallas as pl
from jax.experimental.pallas import tpu as pltpu

Q = 1024
DIM = 16
N = 100000
CH = 1024
NP = 100352
NCH = NP // CH
QB = 128
K = 10
PAD_VAL = 1e15


def _tc_body(xb_ref, yT_ref, ds_ref, di_ref):
    xb = xb_ref[...]
    xn = jnp.sum(xb * xb, axis=1, keepdims=True)

    def chunk(c, carry):
        cd, ci = carry
        yTc = yT_ref[:, pl.ds(c * CH, CH)]
        z = jax.lax.dot_general(
            xb, yTc, (((1,), (0,)), ((), ())),
            preferred_element_type=jnp.float32,
            precision=jax.lax.Precision.HIGHEST)
        yn = jnp.sum(yTc * yTc, axis=0, keepdims=True)
        d2 = xn + yn - 2.0 * z
        gidx = jax.lax.broadcasted_iota(jnp.int32, (QB, CH), 1) + c * CH
        Dm = jnp.concatenate([cd, d2], axis=1)
        Im = jnp.concatenate([ci, gidx], axis=1)
        pos = jax.lax.broadcasted_iota(jnp.int32, (QB, K + CH), 1)
        outs_d, outs_i = [], []
        for _ in range(K):
            m = jnp.min(Dm, axis=1, keepdims=True)
            p = jnp.min(jnp.where(Dm == m, pos, jnp.int32(2**30)),
                        axis=1, keepdims=True)
            sel = pos == p
            iv = jnp.min(jnp.where(sel, Im, jnp.int32(2**31 - 1)),
                         axis=1, keepdims=True)
            outs_d.append(m)
            outs_i.append(iv)
            Dm = jnp.where(sel, jnp.float32(3e38), Dm)
        return (jnp.concatenate(outs_d, axis=1),
                jnp.concatenate(outs_i, axis=1))

    cd0 = jnp.full((QB, K), 3e38, jnp.float32)
    ci0 = jnp.full((QB, K), 2**31 - 1, jnp.int32)
    cd, ci = jax.lax.fori_loop(0, NCH, chunk, (cd0, ci0))
    ds_ref[...] = jnp.sqrt(cd)
    di_ref[...] = ci


def kernel(x, y, k, n_splits):
    del k, n_splits
    yp = jnp.concatenate(
        [y, jnp.full((NP - N, DIM), PAD_VAL, jnp.float32)], axis=0)
    yT = yp.T
    ds, di = pl.pallas_call(
        _tc_body,
        grid=(Q // QB,),
        in_specs=[
            pl.BlockSpec((QB, DIM), lambda i: (i, 0)),
            pl.BlockSpec((DIM, NP), lambda i: (0, 0)),
        ],
        out_specs=[
            pl.BlockSpec((QB, K), lambda i: (i, 0)),
            pl.BlockSpec((QB, K), lambda i: (i, 0)),
        ],
        out_shape=[
            jax.ShapeDtypeStruct((Q, K), jnp.float32),
            jax.ShapeDtypeStruct((Q, K), jnp.int32),
        ],
    )(x, yT)
    return ds, di[..., None]


# speedup vs baseline: 5.2623x; 3.5317x over previous
import jax
import jax.numpy as jnp
from jax import lax
from jax.experimental import pallas as pl
from jax.experimental.pallas import tpu as pltpu
from jax.experimental.pallas import tpu_sc as plsc

Q = 1024
DIM = 16
N = 100000
G = 64
NP = 100352
NG = NP // G
L = 16
NGV = NG // L
NSEL = 16
K = 10
PAD_VAL = 1e15
NC, NS = 2, 16
NW = NC * NS
QPW = Q // NW
BIG = 3e38

CB = 2048
QB = 256


def _gm_body(yb_ref, xT_ref, gmT_ref):
    yb = yb_ref[...]
    z = lax.dot_general(yb, xT_ref[...], (((1,), (0,)), ((), ())),
                        preferred_element_type=jnp.float32,
                        precision=lax.Precision.HIGHEST)
    yn = jnp.sum(yb * yb, axis=1, keepdims=True)
    s = yn - 2.0 * z
    gmT_ref[...] = jnp.min(s.reshape(CB // G, G, QB), axis=1)


def _sc_body(gm_hbm, xb_hbm, ytr_hbm, ds_hbm, di_hbm,
             gm_t, xb_t, sel_t, idx_t, rows_t,
             outd_t, outi_t, sem):
    cid = lax.axis_index("c")
    sid = lax.axis_index("s")
    w = sid * NC + cid

    def per_query(qi, carry):
        q = w * QPW + qi
        pltpu.sync_copy(gm_hbm.at[pl.ds(q * NG, NG)], gm_t)
        pltpu.sync_copy(xb_hbm.at[pl.ds(q * DIM * L, DIM * L)], xb_t)

        iota16 = lax.iota(jnp.int32, 16)
        inf16 = jnp.full((L,), BIG, jnp.float32)
        zero16 = jnp.zeros((L,), jnp.int32)

        def stepA(i, c3):
            td, ti, kth = c3
            v = gm_t[pl.ds(i * L, L)]
            m = jnp.min(v)

            def merge(_):
                ids = iota16 + i * L
                sd, si = plsc.sort_key_val(v, ids)
                rd = jnp.flip(sd, 0)
                ri = jnp.flip(si, 0)
                keep = td <= rd
                nd = jnp.where(keep, td, rd)
                ni = jnp.where(keep, ti, ri)
                nd, ni = plsc.sort_key_val(nd, ni)
                return nd, ni, jnp.max(nd)

            return lax.cond(m < kth, merge, lambda _: c3, None)

        tdA, tiA, _ = lax.fori_loop(
            0, NGV, stepA, (inf16, zero16, jnp.float32(BIG)))
        sel_t[...] = tiA

        for j in range(8):
            idx_t[pl.ds(j * L, L)] = tiA + j * NG
        pltpu.async_copy(ytr_hbm.at[idx_t], rows_t, sem).wait()

        def stepB(t, c3):
            btd, bti, kth = c3
            s_ = t // 4
            v_ = t % 4
            sq = []
            for j in range(DIM):
                row = (j % 8) * L + s_
                col = (j // 8) * G + v_ * L
                yv = rows_t[row, pl.ds(col, L)]
                d = yv - xb_t[pl.ds(j * L, L)]
                sq.append(d * d)
            a = [sq[i] + sq[i + 8] for i in range(8)]
            b = [a[i] + a[i + 4] for i in range(4)]
            c2 = [b[i] + b[i + 2] for i in range(2)]
            d2v = c2[0] + c2[1]
            m = jnp.min(d2v)

            def merge(_):
                gid = plsc.load_gather(sel_t, [jnp.full((L,), s_, jnp.int32)])
                cidx = gid * G + v_ * L + iota16
                sd, si = plsc.sort_key_val(d2v, cidx)
                rd = jnp.flip(sd, 0)
                ri = jnp.flip(si, 0)
                keep = btd <= rd
                nd = jnp.where(keep, btd, rd)
                ni = jnp.where(keep, bti, ri)
                nd, ni = plsc.sort_key_val(nd, ni)
                return nd, ni, jnp.max(nd)

            return lax.cond(m < kth, merge, lambda _: c3, None)

        btd, bti, _ = lax.fori_loop(
            0, NSEL * (G // L), stepB, (inf16, zero16, jnp.float32(BIG)))

        u = plsc.bitcast(btd, jnp.int32)
        y0 = plsc.bitcast((u >> 1) + 0x1FBD1DF6, jnp.float32)
        y1 = 0.5 * (y0 + btd / y0)
        y2 = 0.5 * (y1 + btd / y1)
        y3 = 0.5 * (y2 + btd / y2)
        outd_t[...] = y3
        outi_t[...] = bti
        pltpu.sync_copy(outd_t, ds_hbm.at[pl.ds(q * L, L)])
        pltpu.sync_copy(outi_t, di_hbm.at[pl.ds(q * L, L)])
        return carry

    lax.fori_loop(0, QPW, per_query, 0)


def kernel(x, y, k, n_splits):
    del k, n_splits
    yp = jnp.concatenate(
        [y, jnp.full((NP - N, DIM), PAD_VAL, jnp.float32)], axis=0)
    xT = x.T
    gmT = pl.pallas_call(
        _gm_body,
        grid=(NP // CB, Q // QB),
        in_specs=[
            pl.BlockSpec((CB, DIM), lambda i, j: (i, 0)),
            pl.BlockSpec((DIM, QB), lambda i, j: (0, j)),
        ],
        out_specs=pl.BlockSpec((CB // G, QB), lambda i, j: (i, j)),
        out_shape=jax.ShapeDtypeStruct((NG, Q), jnp.float32),
    )(yp, xT)
    gm = gmT.T
    ypT = yp.T
    ytr = jnp.concatenate(
        [ypT[:8].reshape(8, NG, G), ypT[8:].reshape(8, NG, G)],
        axis=-1).reshape(8 * NG, 2 * G)

    sc_call = pl.kernel(
        _sc_body,
        out_type=[
            jax.ShapeDtypeStruct((Q * L,), jnp.float32),
            jax.ShapeDtypeStruct((Q * L,), jnp.int32),
        ],
        mesh=plsc.VectorSubcoreMesh(core_axis_name="c", subcore_axis_name="s"),
        compiler_params=pltpu.CompilerParams(needs_layout_passes=False),
        scratch_types=[
            pltpu.VMEM((NG,), jnp.float32),
            pltpu.VMEM((DIM * L,), jnp.float32),
            pltpu.VMEM((L,), jnp.int32),
            pltpu.VMEM((8 * L,), jnp.int32),
            pltpu.VMEM((8 * L, 2 * G), jnp.float32),
            pltpu.VMEM((L,), jnp.float32),
            pltpu.VMEM((L,), jnp.int32),
            pltpu.SemaphoreType.DMA,
        ],
    )
    xbB = jnp.broadcast_to(x.reshape(Q * DIM, 1), (Q * DIM, L)).reshape(-1)
    ds1, di1 = sc_call(gm.reshape(-1), xbB, ytr)
    ds_pad = ds1.reshape(Q, L)
    di_pad = di1.reshape(Q, L)
    return ds_pad[:, :K], di_pad[:, :K, None]


# speedup vs baseline: 5.6555x; 1.0747x over previous
import jax
import jax.numpy as jnp
from jax import lax
from jax.experimental import pallas as pl
from jax.experimental.pallas import tpu as pltpu
from jax.experimental.pallas import tpu_sc as plsc

Q = 1024
DIM = 16
N = 100000
G = 64
NP = 106496
NG = NP // G
L = 16
NGV = NG // L
NSEL = 16
K = 10
PAD_VAL = 1e15
NC, NS = 2, 16
NW = NC * NS
QPW = Q // NW
BIG = 3e38

CB = 8192
QB = 512


def _gm_body(yb_ref, xT_ref, gm_ref):
    yb = yb_ref[...]
    z = lax.dot_general(yb, xT_ref[...], (((1,), (0,)), ((), ())),
                        preferred_element_type=jnp.float32,
                        precision=lax.Precision.HIGHEST)
    yn = jnp.sum(yb * yb, axis=1, keepdims=True)
    s = yn - 2.0 * z
    r = jnp.min(s.reshape(CB // G, G, QB), axis=1)
    gm_ref[...] = r.T


def _sc_body(gm_hbm, xb_hbm, yg_hbm, out_hbm,
             gm_t, xb_t, sel_t, rows_t, out_t, sem):
    cid = lax.axis_index("c")
    sid = lax.axis_index("s")
    w = sid * NC + cid

    def per_query(qi, carry):
        q = w * QPW + qi
        pltpu.sync_copy(gm_hbm.at[pl.ds(q * NG, NG)], gm_t)
        pltpu.sync_copy(xb_hbm.at[pl.ds(q * DIM * L, DIM * L)], xb_t)

        iota16 = lax.iota(jnp.int32, 16)
        ioD = iota16 * DIM
        inf16 = jnp.full((L,), BIG, jnp.float32)
        zero16 = jnp.zeros((L,), jnp.int32)

        def stepA(i, c3):
            td, ti, kth = c3
            v = gm_t[pl.ds(i * L, L)]
            m = jnp.min(v)

            def merge(_):
                ids = iota16 + i * L
                sd, si = plsc.sort_key_val(v, ids)
                rd = jnp.flip(sd, 0)
                ri = jnp.flip(si, 0)
                keep = td <= rd
                nd = jnp.where(keep, td, rd)
                ni = jnp.where(keep, ti, ri)
                nd, ni = plsc.sort_key_val(nd, ni)
                return nd, ni, jnp.max(nd)

            return lax.cond(m < kth, merge, lambda _: c3, None)

        tdA, tiA, _ = lax.fori_loop(
            0, NGV, stepA, (inf16, zero16, jnp.float32(BIG)))
        sel_t[...] = tiA

        pltpu.async_copy(yg_hbm.at[sel_t], rows_t, sem).wait()

        def stepB(t, c3):
            btd, bti, kth = c3
            s_ = t // 4
            v_ = t % 4
            rowsplat = jnp.full((L,), s_, jnp.int32)
            colbase = ioD + v_ * (L * DIM)
            sq = []
            for j in range(DIM):
                yv = plsc.load_gather(rows_t, [rowsplat, colbase + j])
                d = yv - xb_t[pl.ds(j * L, L)]
                sq.append(d * d)
            a = [sq[i] + sq[i + 8] for i in range(8)]
            b = [a[i] + a[i + 4] for i in range(4)]
            c2 = [b[i] + b[i + 2] for i in range(2)]
            d2v = c2[0] + c2[1]
            m = jnp.min(d2v)

            def merge(_):
                gid = plsc.load_gather(sel_t, [rowsplat])
                cidx = gid * G + v_ * L + iota16
                sd, si = plsc.sort_key_val(d2v, cidx)
                rd = jnp.flip(sd, 0)
                ri = jnp.flip(si, 0)
                keep = btd <= rd
                nd = jnp.where(keep, btd, rd)
                ni = jnp.where(keep, bti, ri)
                nd, ni = plsc.sort_key_val(nd, ni)
                return nd, ni, jnp.max(nd)

            return lax.cond(m < kth, merge, lambda _: c3, None)

        btd, bti, _ = lax.fori_loop(
            0, NSEL * (G // L), stepB, (inf16, zero16, jnp.float32(BIG)))

        u = plsc.bitcast(btd, jnp.int32)
        y0 = plsc.bitcast((u >> 1) + 0x1FBD1DF6, jnp.float32)
        y1 = 0.5 * (y0 + btd / y0)
        y2 = 0.5 * (y1 + btd / y1)
        y3 = 0.5 * (y2 + btd / y2)
        out_t[pl.ds(0, L)] = plsc.bitcast(y3, jnp.int32)
        out_t[pl.ds(L, L)] = bti
        pltpu.sync_copy(out_t, out_hbm.at[pl.ds(q * 2 * L, 2 * L)])
        return carry

    lax.fori_loop(0, QPW, per_query, 0)


def kernel(x, y, k, n_splits):
    del k, n_splits
    yp = jnp.concatenate(
        [y, jnp.full((NP - N, DIM), PAD_VAL, jnp.float32)], axis=0)
    gm = pl.pallas_call(
        _gm_body,
        grid=(NP // CB, Q // QB),
        in_specs=[
            pl.BlockSpec((CB, DIM), lambda i, j: (i, 0)),
            pl.BlockSpec((DIM, QB), lambda i, j: (0, j)),
        ],
        out_specs=pl.BlockSpec((QB, CB // G), lambda i, j: (j, i)),
        out_shape=jax.ShapeDtypeStruct((Q, NG), jnp.float32),
    )(yp, x.T)
    yg = yp.reshape(NG, G * DIM)
    xbB = jnp.broadcast_to(x.reshape(Q * DIM, 1), (Q * DIM, L)).reshape(-1)

    sc_call = pl.kernel(
        _sc_body,
        out_type=jax.ShapeDtypeStruct((Q * 2 * L,), jnp.int32),
        mesh=plsc.VectorSubcoreMesh(core_axis_name="c", subcore_axis_name="s"),
        compiler_params=pltpu.CompilerParams(needs_layout_passes=False),
        scratch_types=[
            pltpu.VMEM((NG,), jnp.float32),
            pltpu.VMEM((DIM * L,), jnp.float32),
            pltpu.VMEM((NSEL,), jnp.int32),
            pltpu.VMEM((NSEL, G * DIM), jnp.float32),
            pltpu.VMEM((2 * L,), jnp.int32),
            pltpu.SemaphoreType.DMA,
        ],
    )
    out = sc_call(gm.reshape(-1), xbB, yg).reshape(Q, 2 * L)
    ds = lax.bitcast_convert_type(out[:, :K], jnp.float32)
    di = out[:, L:L + K]
    return ds, di[..., None]


# speedup vs baseline: 6.0405x; 1.0681x over previous
import jax
import jax.numpy as jnp
from jax import lax
from jax.experimental import pallas as pl
from jax.experimental.pallas import tpu as pltpu
from jax.experimental.pallas import tpu_sc as plsc

Q = 1024
DIM = 16
N = 100000
G = 64
NP = 106496
NG = NP // G
L = 16
NGV = NG // L
NSEL = 16
K = 10
PAD_VAL = 1e15
NC, NS = 2, 16
NW = NC * NS
QPW = Q // NW
BIG = 3e38
CMBW = NG + DIM * L

CB = 8192
QB = 512


def _gm_body(yb_ref, xT_ref, gm_ref):
    yb = yb_ref[...]
    z = lax.dot_general(yb, xT_ref[...], (((1,), (0,)), ((), ())),
                        preferred_element_type=jnp.float32,
                        precision=lax.Precision.HIGHEST)
    yn = jnp.sum(yb * yb, axis=1, keepdims=True)
    s = yn - 2.0 * z
    r = jnp.min(s.reshape(CB // G, G, QB), axis=1)
    gm_ref[...] = r.T


def _sc_body(cmb_hbm, yg_hbm, out_hbm,
             cmb_t0, cmb_t1, sel_t0, sel_t1, rows_t0, rows_t1,
             out_t, sem0, sem1, semc):
    cid = lax.axis_index("c")
    sid = lax.axis_index("s")
    w = sid * NC + cid

    iota16 = lax.iota(jnp.int32, 16)
    ioD = iota16 * DIM
    inf16 = jnp.full((L,), BIG, jnp.float32)
    zero16 = jnp.zeros((L,), jnp.int32)

    def phase_a(cmb_t):
        def stepA(i, c3):
            td, ti, kth = c3
            v = cmb_t[pl.ds(i * L, L)]
            m = jnp.min(v)

            def merge(_):
                ids = iota16 + i * L
                sd, si = plsc.sort_key_val(v, ids)
                rd = jnp.flip(sd, 0)
                ri = jnp.flip(si, 0)
                keep = td <= rd
                nd = jnp.where(keep, td, rd)
                ni = jnp.where(keep, ti, ri)
                nd, ni = plsc.sort_key_val(nd, ni)
                return nd, ni, jnp.max(nd)

            return lax.cond(m < kth, merge, lambda _: c3, None)

        _, tiA, _ = lax.fori_loop(
            0, NGV, stepA, (inf16, zero16, jnp.float32(BIG)))
        return tiA

    def phase_b(cmb_t, sel_t, rows_t):
        def stepB(t, c3):
            btd, bti, kth = c3
            s_ = t // 4
            v_ = t % 4
            rowsplat = jnp.full((L,), s_, jnp.int32)
            colbase = ioD + v_ * (L * DIM)
            sq = []
            for j in range(DIM):
                yv = plsc.load_gather(rows_t, [rowsplat, colbase + j])
                d = yv - cmb_t[pl.ds(NG + j * L, L)]
                sq.append(d * d)
            a = [sq[i] + sq[i + 8] for i in range(8)]
            b = [a[i] + a[i + 4] for i in range(4)]
            c2 = [b[i] + b[i + 2] for i in range(2)]
            d2v = c2[0] + c2[1]
            m = jnp.min(d2v)

            def merge(_):
                gid = plsc.load_gather(sel_t, [rowsplat])
                cidx = gid * G + v_ * L + iota16
                sd, si = plsc.sort_key_val(d2v, cidx)
                rd = jnp.flip(sd, 0)
                ri = jnp.flip(si, 0)
                keep = btd <= rd
                nd = jnp.where(keep, btd, rd)
                ni = jnp.where(keep, bti, ri)
                nd, ni = plsc.sort_key_val(nd, ni)
                return nd, ni, jnp.max(nd)

            return lax.cond(m < kth, merge, lambda _: c3, None)

        btd, bti, _ = lax.fori_loop(
            0, NSEL * (G // L), stepB, (inf16, zero16, jnp.float32(BIG)))
        return btd, bti

    def emit(q, btd, bti):
        u = plsc.bitcast(btd, jnp.int32)
        y0 = plsc.bitcast((u >> 1) + 0x1FBD1DF6, jnp.float32)
        y1 = 0.5 * (y0 + btd / y0)
        y2 = 0.5 * (y1 + btd / y1)
        y3 = 0.5 * (y2 + btd / y2)
        out_t[pl.ds(0, L)] = plsc.bitcast(y3, jnp.int32)
        out_t[pl.ds(L, L)] = bti
        pltpu.sync_copy(out_t, out_hbm.at[pl.ds(q * 2 * L, 2 * L)])

    def per_pair(p, carry):
        q0 = w * QPW + 2 * p
        q1 = q0 + 1
        pltpu.sync_copy(cmb_hbm.at[pl.ds(q0 * CMBW, CMBW)], cmb_t0)
        sel_t0[...] = phase_a(cmb_t0)
        cp0 = pltpu.async_copy(yg_hbm.at[sel_t0], rows_t0, sem0)
        pltpu.sync_copy(cmb_hbm.at[pl.ds(q1 * CMBW, CMBW)], cmb_t1)
        sel_t1[...] = phase_a(cmb_t1)
        cp1 = pltpu.async_copy(yg_hbm.at[sel_t1], rows_t1, sem1)
        cp0.wait()
        btd, bti = phase_b(cmb_t0, sel_t0, rows_t0)
        emit(q0, btd, bti)
        cp1.wait()
        btd, bti = phase_b(cmb_t1, sel_t1, rows_t1)
        emit(q1, btd, bti)
        return carry

    lax.fori_loop(0, QPW // 2, per_pair, 0)


def kernel(x, y, k, n_splits):
    del k, n_splits
    yp = jnp.concatenate(
        [y, jnp.full((NP - N, DIM), PAD_VAL, jnp.float32)], axis=0)
    gm = pl.pallas_call(
        _gm_body,
        grid=(NP // CB, Q // QB),
        in_specs=[
            pl.BlockSpec((CB, DIM), lambda i, j: (i, 0)),
            pl.BlockSpec((DIM, QB), lambda i, j: (0, j)),
        ],
        out_specs=pl.BlockSpec((QB, CB // G), lambda i, j: (j, i)),
        out_shape=jax.ShapeDtypeStruct((Q, NG), jnp.float32),
    )(yp, x.T)
    yg = yp.reshape(NG, G * DIM)
    xbB = jnp.broadcast_to(x.reshape(Q * DIM, 1), (Q * DIM, L))
    cmb = jnp.concatenate([gm, xbB.reshape(Q, DIM * L)], axis=1)

    sc_call = pl.kernel(
        _sc_body,
        out_type=jax.ShapeDtypeStruct((Q * 2 * L,), jnp.int32),
        mesh=plsc.VectorSubcoreMesh(core_axis_name="c", subcore_axis_name="s"),
        compiler_params=pltpu.CompilerParams(needs_layout_passes=False),
        scratch_types=[
            pltpu.VMEM((CMBW,), jnp.float32),
            pltpu.VMEM((CMBW,), jnp.float32),
            pltpu.VMEM((NSEL,), jnp.int32),
            pltpu.VMEM((NSEL,), jnp.int32),
            pltpu.VMEM((NSEL, G * DIM), jnp.float32),
            pltpu.VMEM((NSEL, G * DIM), jnp.float32),
            pltpu.VMEM((2 * L,), jnp.int32),
            pltpu.SemaphoreType.DMA,
            pltpu.SemaphoreType.DMA,
            pltpu.SemaphoreType.DMA,
        ],
    )
    out = sc_call(cmb.reshape(-1), yg).reshape(Q, 2 * L)
    ds = lax.bitcast_convert_type(out[:, :K], jnp.float32)
    di = out[:, L:L + K]
    return ds, di[..., None]


# speedup vs baseline: 7.7824x; 1.2884x over previous
import jax
import jax.numpy as jnp
from jax import lax
from jax.experimental import pallas as pl
from jax.experimental.pallas import tpu as pltpu
from jax.experimental.pallas import tpu_sc as plsc

Q = 1024
DIM = 16
N = 100000
G = 64
NP = 106496
NG = NP // G
L = 16
NGV = NG // L
NSEL = 16
K = 10
PAD_VAL = 1e15
NC, NS = 2, 16
NW = NC * NS
QPW = Q // NW
BIG = 3e38
CMBW = NG + DIM * L

CB = 8192
QB = 512


def _gm_body(yb_ref, xT_ref, gm_ref):
    yb = yb_ref[...]
    xT = xT_ref[...]
    ybh = yb.astype(jnp.bfloat16)
    ybl = (yb - ybh.astype(jnp.float32)).astype(jnp.bfloat16)
    xh = xT.astype(jnp.bfloat16)
    xl = (xT - xh.astype(jnp.float32)).astype(jnp.bfloat16)
    dn = (((1,), (0,)), ((), ()))
    z = (lax.dot_general(ybh, xh, dn, preferred_element_type=jnp.float32)
         + (lax.dot_general(ybh, xl, dn, preferred_element_type=jnp.float32)
            + lax.dot_general(ybl, xh, dn,
                              preferred_element_type=jnp.float32)))
    yn = jnp.sum(yb * yb, axis=1, keepdims=True)
    s = yn - 2.0 * z
    r = jnp.min(s.reshape(CB // G, G, QB), axis=1)
    gm_ref[...] = r.T


def _sc_body(cmb_hbm, yg_hbm, out_hbm,
             cmb_t0, cmb_t1, sel_t0, sel_t1, rows_t0, rows_t1,
             out_t, sem0, sem1, semc):
    cid = lax.axis_index("c")
    sid = lax.axis_index("s")
    w = sid * NC + cid

    iota16 = lax.iota(jnp.int32, 16)
    ioD = iota16 * DIM
    inf16 = jnp.full((L,), BIG, jnp.float32)
    zero16 = jnp.zeros((L,), jnp.int32)

    def phase_a(cmb_t):
        def stepA(i, c3):
            td, ti, kth = c3
            v = cmb_t[pl.ds(i * L, L)]
            m = jnp.min(v)

            def merge(_):
                ids = iota16 + i * L
                sd, si = plsc.sort_key_val(v, ids)
                rd = jnp.flip(sd, 0)
                ri = jnp.flip(si, 0)
                keep = td <= rd
                nd = jnp.where(keep, td, rd)
                ni = jnp.where(keep, ti, ri)
                nd, ni = plsc.sort_key_val(nd, ni)
                return nd, ni, jnp.max(nd)

            return lax.cond(m < kth, merge, lambda _: c3, None)

        _, tiA, _ = lax.fori_loop(
            0, NGV, stepA, (inf16, zero16, jnp.float32(BIG)))
        return tiA

    def phase_b(cmb_t, sel_t, rows_t):
        def stepB(t, c3):
            btd, bti, kth = c3
            s_ = t // 4
            v_ = t % 4
            rowsplat = jnp.full((L,), s_, jnp.int32)
            colbase = ioD + v_ * (L * DIM)
            sq = []
            for j in range(DIM):
                yv = plsc.load_gather(rows_t, [rowsplat, colbase + j])
                d = yv - cmb_t[pl.ds(NG + j * L, L)]
                sq.append(d * d)
            a = [sq[i] + sq[i + 8] for i in range(8)]
            b = [a[i] + a[i + 4] for i in range(4)]
            c2 = [b[i] + b[i + 2] for i in range(2)]
            d2v = c2[0] + c2[1]
            m = jnp.min(d2v)

            def merge(_):
                gid = plsc.load_gather(sel_t, [rowsplat])
                cidx = gid * G + v_ * L + iota16
                sd, si = plsc.sort_key_val(d2v, cidx)
                rd = jnp.flip(sd, 0)
                ri = jnp.flip(si, 0)
                keep = btd <= rd
                nd = jnp.where(keep, btd, rd)
                ni = jnp.where(keep, bti, ri)
                nd, ni = plsc.sort_key_val(nd, ni)
                return nd, ni, jnp.max(nd)

            return lax.cond(m < kth, merge, lambda _: c3, None)

        btd, bti, _ = lax.fori_loop(
            0, NSEL * (G // L), stepB, (inf16, zero16, jnp.float32(BIG)))
        return btd, bti

    def emit(q, btd, bti):
        u = plsc.bitcast(btd, jnp.int32)
        y0 = plsc.bitcast((u >> 1) + 0x1FBD1DF6, jnp.float32)
        y1 = 0.5 * (y0 + btd / y0)
        y2 = 0.5 * (y1 + btd / y1)
        y3 = 0.5 * (y2 + btd / y2)
        out_t[pl.ds(0, L)] = plsc.bitcast(y3, jnp.int32)
        out_t[pl.ds(L, L)] = bti
        pltpu.sync_copy(out_t, out_hbm.at[pl.ds(q * 2 * L, 2 * L)])

    def per_pair(p, carry):
        q0 = w * QPW + 2 * p
        q1 = q0 + 1
        pltpu.sync_copy(cmb_hbm.at[pl.ds(q0 * CMBW, CMBW)], cmb_t0)
        sel_t0[...] = phase_a(cmb_t0)
        cp0 = pltpu.async_copy(yg_hbm.at[sel_t0], rows_t0, sem0)
        pltpu.sync_copy(cmb_hbm.at[pl.ds(q1 * CMBW, CMBW)], cmb_t1)
        sel_t1[...] = phase_a(cmb_t1)
        cp1 = pltpu.async_copy(yg_hbm.at[sel_t1], rows_t1, sem1)
        cp0.wait()
        btd, bti = phase_b(cmb_t0, sel_t0, rows_t0)
        emit(q0, btd, bti)
        cp1.wait()
        btd, bti = phase_b(cmb_t1, sel_t1, rows_t1)
        emit(q1, btd, bti)
        return carry

    lax.fori_loop(0, QPW // 2, per_pair, 0)


def kernel(x, y, k, n_splits):
    del k, n_splits
    yp = jnp.concatenate(
        [y, jnp.full((NP - N, DIM), PAD_VAL, jnp.float32)], axis=0)
    gm = pl.pallas_call(
        _gm_body,
        grid=(NP // CB, Q // QB),
        in_specs=[
            pl.BlockSpec((CB, DIM), lambda i, j: (i, 0)),
            pl.BlockSpec((DIM, QB), lambda i, j: (0, j)),
        ],
        out_specs=pl.BlockSpec((QB, CB // G), lambda i, j: (j, i)),
        out_shape=jax.ShapeDtypeStruct((Q, NG), jnp.float32),
    )(yp, x.T)
    yg = yp.reshape(NG, G * DIM)
    xbB = jnp.broadcast_to(x.reshape(Q * DIM, 1), (Q * DIM, L))
    cmb = jnp.concatenate([gm, xbB.reshape(Q, DIM * L)], axis=1)

    sc_call = pl.kernel(
        _sc_body,
        out_type=jax.ShapeDtypeStruct((Q * 2 * L,), jnp.int32),
        mesh=plsc.VectorSubcoreMesh(core_axis_name="c", subcore_axis_name="s"),
        compiler_params=pltpu.CompilerParams(needs_layout_passes=False),
        scratch_types=[
            pltpu.VMEM((CMBW,), jnp.float32),
            pltpu.VMEM((CMBW,), jnp.float32),
            pltpu.VMEM((NSEL,), jnp.int32),
            pltpu.VMEM((NSEL,), jnp.int32),
            pltpu.VMEM((NSEL, G * DIM), jnp.float32),
            pltpu.VMEM((NSEL, G * DIM), jnp.float32),
            pltpu.VMEM((2 * L,), jnp.int32),
            pltpu.SemaphoreType.DMA,
            pltpu.SemaphoreType.DMA,
            pltpu.SemaphoreType.DMA,
        ],
    )
    out = sc_call(cmb.reshape(-1), yg).reshape(Q, 2 * L)
    ds = lax.bitcast_convert_type(out[:, :K], jnp.float32)
    di = out[:, L:L + K]
    return ds, di[..., None]
